# Initial kernel scaffold; baseline (speedup 1.0000x reference)
import jax
import jax.numpy as jnp
from jax.experimental import pallas as pl


def kernel(x, node_attrs, edge_index, atomic_numbers, covalent_radii):
    raise NotImplementedError("write your pallas kernel here")


# speedup vs baseline: 304.6355x; 304.6355x over previous
import functools

import jax
import jax.numpy as jnp
from jax import lax
from jax.experimental import pallas as pl
from jax.experimental.pallas import tpu as pltpu
from jax.experimental.pallas import tpu_sc as plsc

_N_NODES = 100000
_N_EDGES = 6400000
_NUM_ELEMENTS = 10
_N_SPECIES = 119

_NODE_BLOCK = 10000

_NW = 32
_EDGE_PER_W = _N_EDGES // _NW
_CHUNK = 2000
_NCHUNK = _EDGE_PER_W // _CHUNK
_L = 16


def _node_rad_body(attrs_ref, an_ref, cr_ref, out_ref):
    attrs = attrs_ref[...]
    an = an_ref[...]
    cr = cr_ref[...]
    bn = attrs.shape[0]
    ei = lax.broadcasted_iota(jnp.int32, (bn, _NUM_ELEMENTS), 1)
    rowmax = jnp.max(attrs, axis=1, keepdims=True)
    first = jnp.min(
        jnp.where(attrs == rowmax, ei, _NUM_ELEMENTS), axis=1, keepdims=True
    )
    z = jnp.sum(jnp.where(ei == first, an, 0), axis=1, keepdims=True)
    zi = lax.broadcasted_iota(jnp.int32, (bn, _N_SPECIES), 1)
    rad = jnp.sum(jnp.where(zi == z, cr, 0.0), axis=1, keepdims=True)
    out_ref[...] = rad


def _node_rad_call(node_attrs, atomic_numbers, covalent_radii):
    n = node_attrs.shape[0]
    grid = n // _NODE_BLOCK
    return pl.pallas_call(
        _node_rad_body,
        grid=(grid,),
        in_specs=[
            pl.BlockSpec((_NODE_BLOCK, _NUM_ELEMENTS), lambda i: (i, 0)),
            pl.BlockSpec((1, _NUM_ELEMENTS), lambda i: (0, 0)),
            pl.BlockSpec((1, _N_SPECIES), lambda i: (0, 0)),
        ],
        out_specs=pl.BlockSpec((_NODE_BLOCK, 1), lambda i: (i, 0)),
        out_shape=jax.ShapeDtypeStruct((n, 1), jnp.float32),
    )(node_attrs, atomic_numbers.reshape(1, -1), covalent_radii.reshape(1, -1))


def _edge_body(rad_hbm, s_hbm, r_hbm, x_hbm, out_hbm,
               rad_v, sidx_v, ridx_v, x_v, o_v):
    wid = lax.axis_index("s") * 2 + lax.axis_index("c")
    base_w = wid * _EDGE_PER_W
    pltpu.sync_copy(rad_hbm, rad_v)

    def chunk(g, carry):
        base = base_w + g * _CHUNK
        pltpu.sync_copy(s_hbm.at[pl.ds(base, _CHUNK)], sidx_v)
        pltpu.sync_copy(r_hbm.at[pl.ds(base, _CHUNK)], ridx_v)
        pltpu.sync_copy(x_hbm.at[pl.ds(base, _CHUNK)], x_v)

        def step(i, c):
            si = sidx_v[pl.ds(i * _L, _L)]
            ri = ridx_v[pl.ds(i * _L, _L)]
            ra = plsc.load_gather(rad_v, [si])
            rb = plsc.load_gather(rad_v, [ri])
            r0 = ra + rb
            xv = x_v[pl.ds(i * _L, _L)]
            t = (96.0 / 7.0) * (xv / r0) - (100.0 / 7.0)
            s = 1.0 / (1.0 + jnp.exp(-t))
            p0 = 0.75 * r0
            o_v[pl.ds(i * _L, _L)] = p0 + (xv - p0) * s
            return c

        lax.fori_loop(0, _CHUNK // _L, step, 0)
        pltpu.sync_copy(o_v, out_hbm.at[pl.ds(base, _CHUNK)])
        return carry

    lax.fori_loop(0, _NCHUNK, chunk, 0)


def _edge_call(rad, sender, receiver, xf):
    mesh = plsc.VectorSubcoreMesh(core_axis_name="c", subcore_axis_name="s")
    return pl.kernel(
        _edge_body,
        out_type=jax.ShapeDtypeStruct((_N_EDGES,), jnp.float32),
        mesh=mesh,
        scratch_types=[
            pltpu.VMEM((_N_NODES,), jnp.float32),
            pltpu.VMEM((_CHUNK,), jnp.int32),
            pltpu.VMEM((_CHUNK,), jnp.int32),
            pltpu.VMEM((_CHUNK,), jnp.float32),
            pltpu.VMEM((_CHUNK,), jnp.float32),
        ],
        compiler_params=pltpu.CompilerParams(needs_layout_passes=False),
    )(rad, sender, receiver, xf)


def kernel(x, node_attrs, edge_index, atomic_numbers, covalent_radii):
    rad = _node_rad_call(node_attrs, atomic_numbers, covalent_radii)
    out = _edge_call(
        rad.reshape(-1),
        edge_index[0],
        edge_index[1],
        x.reshape(-1),
    )
    return out.reshape(-1, 1)


# speedup vs baseline: 654.7679x; 2.1493x over previous
import functools

import jax
import jax.numpy as jnp
from jax import lax
from jax.experimental import pallas as pl
from jax.experimental.pallas import tpu as pltpu
from jax.experimental.pallas import tpu_sc as plsc

_N_NODES = 100000
_N_EDGES = 6400000
_NUM_ELEMENTS = 10
_N_SPECIES = 119

_NODE_BLOCK = 10000

_NW = 32
_EDGE_PER_W = _N_EDGES // _NW
_CHUNK = 4000
_NCHUNK = _EDGE_PER_W // _CHUNK
_L = 16
_UNROLL = 5


def _node_rad_body(attrs_ref, an_ref, cr_ref, out_ref):
    attrs = attrs_ref[...]
    an = an_ref[...]
    cr = cr_ref[...]
    bn = attrs.shape[0]
    ei = lax.broadcasted_iota(jnp.int32, (bn, _NUM_ELEMENTS), 1)
    rowmax = jnp.max(attrs, axis=1, keepdims=True)
    first = jnp.min(
        jnp.where(attrs == rowmax, ei, _NUM_ELEMENTS), axis=1, keepdims=True
    )
    z = jnp.sum(jnp.where(ei == first, an, 0), axis=1, keepdims=True)
    zi = lax.broadcasted_iota(jnp.int32, (bn, _N_SPECIES), 1)
    rad = jnp.sum(jnp.where(zi == z, cr, 0.0), axis=1, keepdims=True)
    out_ref[...] = rad


def _node_rad_call(node_attrs, atomic_numbers, covalent_radii):
    n = node_attrs.shape[0]
    grid = n // _NODE_BLOCK
    return pl.pallas_call(
        _node_rad_body,
        grid=(grid,),
        in_specs=[
            pl.BlockSpec((_NODE_BLOCK, _NUM_ELEMENTS), lambda i: (i, 0)),
            pl.BlockSpec((1, _NUM_ELEMENTS), lambda i: (0, 0)),
            pl.BlockSpec((1, _N_SPECIES), lambda i: (0, 0)),
        ],
        out_specs=pl.BlockSpec((_NODE_BLOCK, 1), lambda i: (i, 0)),
        out_shape=jax.ShapeDtypeStruct((n, 1), jnp.float32),
    )(node_attrs, atomic_numbers.reshape(1, -1), covalent_radii.reshape(1, -1))


def _edge_body(rad_hbm, s_hbm, r_hbm, x_hbm, out_hbm,
               rad_v, sidx_v, ridx_v, x_v, o_v):
    wid = lax.axis_index("s") * 2 + lax.axis_index("c")
    base_w = wid * _EDGE_PER_W
    pltpu.sync_copy(rad_hbm, rad_v)

    def chunk(g, carry):
        base = base_w + g * _CHUNK
        pltpu.sync_copy(s_hbm.at[pl.ds(base, _CHUNK)], sidx_v)
        pltpu.sync_copy(r_hbm.at[pl.ds(base, _CHUNK)], ridx_v)
        pltpu.sync_copy(x_hbm.at[pl.ds(base, _CHUNK)], x_v)

        @plsc.parallel_loop(0, _CHUNK, step=_L, unroll=_UNROLL)
        def step(i):
            si = sidx_v[pl.ds(i, _L)]
            ri = ridx_v[pl.ds(i, _L)]
            ra = plsc.load_gather(rad_v, [si])
            rb = plsc.load_gather(rad_v, [ri])
            r0 = ra + rb
            xv = x_v[pl.ds(i, _L)]
            t = (100.0 / 7.0) - (96.0 / 7.0) * (xv / r0)
            s = 1.0 / (1.0 + jnp.exp(t))
            p0 = 0.75 * r0
            o_v[pl.ds(i, _L)] = p0 + (xv - p0) * s
        pltpu.sync_copy(o_v, out_hbm.at[pl.ds(base, _CHUNK)])
        return carry

    lax.fori_loop(0, _NCHUNK, chunk, 0)


def _edge_call(rad, sender, receiver, xf):
    mesh = plsc.VectorSubcoreMesh(core_axis_name="c", subcore_axis_name="s")
    return pl.kernel(
        _edge_body,
        out_type=jax.ShapeDtypeStruct((_N_EDGES,), jnp.float32),
        mesh=mesh,
        scratch_types=[
            pltpu.VMEM((_N_NODES,), jnp.float32),
            pltpu.VMEM((_CHUNK,), jnp.int32),
            pltpu.VMEM((_CHUNK,), jnp.int32),
            pltpu.VMEM((_CHUNK,), jnp.float32),
            pltpu.VMEM((_CHUNK,), jnp.float32),
        ],
        compiler_params=pltpu.CompilerParams(needs_layout_passes=False),
    )(rad, sender, receiver, xf)


def kernel(x, node_attrs, edge_index, atomic_numbers, covalent_radii):
    rad = _node_rad_call(node_attrs, atomic_numbers, covalent_radii)
    out = _edge_call(
        rad.reshape(-1),
        edge_index[0],
        edge_index[1],
        x.reshape(-1),
    )
    return out.reshape(-1, 1)


# speedup vs baseline: 702.6180x; 1.0731x over previous
import functools

import jax
import jax.numpy as jnp
from jax import lax
from jax.experimental import pallas as pl
from jax.experimental.pallas import tpu as pltpu
from jax.experimental.pallas import tpu_sc as plsc

_N_NODES = 100000
_N_EDGES = 6400000
_NUM_ELEMENTS = 10
_N_SPECIES = 119

_NODE_BLOCK = 10000

_NW = 32
_EDGE_PER_W = _N_EDGES // _NW
_CHUNK = 4000
_NCHUNK = _EDGE_PER_W // _CHUNK
_L = 16
_UNROLL = 5


def _node_rad_body(attrs_ref, an_ref, cr_ref, out_ref):
    attrs = attrs_ref[...]
    an = an_ref[...]
    cr = cr_ref[...]
    bn = attrs.shape[0]
    ei = lax.broadcasted_iota(jnp.int32, (bn, _NUM_ELEMENTS), 1)
    rowmax = jnp.max(attrs, axis=1, keepdims=True)
    first = jnp.min(
        jnp.where(attrs == rowmax, ei, _NUM_ELEMENTS), axis=1, keepdims=True
    )
    z = jnp.sum(jnp.where(ei == first, an, 0), axis=1, keepdims=True)
    zi = lax.broadcasted_iota(jnp.int32, (bn, _N_SPECIES), 1)
    rad = jnp.sum(jnp.where(zi == z, cr, 0.0), axis=1, keepdims=True)
    out_ref[...] = rad


def _node_rad_call(node_attrs, atomic_numbers, covalent_radii):
    n = node_attrs.shape[0]
    grid = n // _NODE_BLOCK
    return pl.pallas_call(
        _node_rad_body,
        grid=(grid,),
        in_specs=[
            pl.BlockSpec((_NODE_BLOCK, _NUM_ELEMENTS), lambda i: (i, 0)),
            pl.BlockSpec((1, _NUM_ELEMENTS), lambda i: (0, 0)),
            pl.BlockSpec((1, _N_SPECIES), lambda i: (0, 0)),
        ],
        out_specs=pl.BlockSpec((_NODE_BLOCK, 1), lambda i: (i, 0)),
        out_shape=jax.ShapeDtypeStruct((n, 1), jnp.float32),
    )(node_attrs, atomic_numbers.reshape(1, -1), covalent_radii.reshape(1, -1))


def _edge_body(rad_hbm, ei_hbm, x_hbm, out_hbm,
               rad_v, sidx_v, ridx_v, x_v, o_v):
    wid = lax.axis_index("s") * 2 + lax.axis_index("c")
    base_w = wid * _EDGE_PER_W
    pltpu.sync_copy(rad_hbm, rad_v)

    def chunk(g, carry):
        base = base_w + g * _CHUNK
        pltpu.sync_copy(ei_hbm.at[pl.ds(base, _CHUNK)], sidx_v)
        pltpu.sync_copy(ei_hbm.at[pl.ds(_N_EDGES + base, _CHUNK)], ridx_v)
        pltpu.sync_copy(x_hbm.at[pl.ds(base, _CHUNK)], x_v)

        @plsc.parallel_loop(0, _CHUNK, step=_L, unroll=_UNROLL)
        def step(i):
            si = sidx_v[pl.ds(i, _L)]
            ri = ridx_v[pl.ds(i, _L)]
            ra = plsc.load_gather(rad_v, [si])
            rb = plsc.load_gather(rad_v, [ri])
            r0 = ra + rb
            xv = x_v[pl.ds(i, _L)]
            t = (100.0 / 7.0) - (96.0 / 7.0) * (xv / r0)
            s = 1.0 / (1.0 + jnp.exp(t))
            p0 = 0.75 * r0
            o_v[pl.ds(i, _L)] = p0 + (xv - p0) * s
        pltpu.sync_copy(o_v, out_hbm.at[pl.ds(base, _CHUNK)])
        return carry

    lax.fori_loop(0, _NCHUNK, chunk, 0)


def _edge_call(rad, edge_index, xf):
    mesh = plsc.VectorSubcoreMesh(core_axis_name="c", subcore_axis_name="s")
    return pl.kernel(
        _edge_body,
        out_type=jax.ShapeDtypeStruct((_N_EDGES,), jnp.float32),
        mesh=mesh,
        scratch_types=[
            pltpu.VMEM((_N_NODES,), jnp.float32),
            pltpu.VMEM((_CHUNK,), jnp.int32),
            pltpu.VMEM((_CHUNK,), jnp.int32),
            pltpu.VMEM((_CHUNK,), jnp.float32),
            pltpu.VMEM((_CHUNK,), jnp.float32),
        ],
        compiler_params=pltpu.CompilerParams(needs_layout_passes=False),
    )(rad, edge_index, xf)


def kernel(x, node_attrs, edge_index, atomic_numbers, covalent_radii):
    rad = _node_rad_call(node_attrs, atomic_numbers, covalent_radii)
    out = _edge_call(rad.reshape(-1), edge_index.reshape(-1), x.reshape(-1))
    return out.reshape(-1, 1)


# speedup vs baseline: 964.6852x; 1.3730x over previous
import functools

import jax
import jax.numpy as jnp
from jax import lax
from jax.experimental import pallas as pl
from jax.experimental.pallas import tpu as pltpu
from jax.experimental.pallas import tpu_sc as plsc

_N_NODES = 100000
_N_EDGES = 6400000
_NUM_ELEMENTS = 10
_N_SPECIES = 119

_NODE_BLOCK = 10000

_NW = 32
_EDGE_PER_W = _N_EDGES // _NW
_CHUNK = 2000
_NCHUNK = _EDGE_PER_W // _CHUNK
_L = 16
_UNROLL = 5


def _node_rad_body(attrs_ref, an_ref, cr_ref, out_ref):
    attrs = attrs_ref[...]
    an = an_ref[...]
    cr = cr_ref[...]
    bn = attrs.shape[0]
    ei = lax.broadcasted_iota(jnp.int32, (bn, _NUM_ELEMENTS), 1)
    rowmax = jnp.max(attrs, axis=1, keepdims=True)
    first = jnp.min(
        jnp.where(attrs == rowmax, ei, _NUM_ELEMENTS), axis=1, keepdims=True
    )
    z = jnp.sum(jnp.where(ei == first, an, 0), axis=1, keepdims=True)
    zi = lax.broadcasted_iota(jnp.int32, (bn, _N_SPECIES), 1)
    rad = jnp.sum(jnp.where(zi == z, cr, 0.0), axis=1, keepdims=True)
    out_ref[...] = rad


def _node_rad_call(node_attrs, atomic_numbers, covalent_radii):
    n = node_attrs.shape[0]
    grid = n // _NODE_BLOCK
    return pl.pallas_call(
        _node_rad_body,
        grid=(grid,),
        in_specs=[
            pl.BlockSpec((_NODE_BLOCK, _NUM_ELEMENTS), lambda i: (i, 0)),
            pl.BlockSpec((1, _NUM_ELEMENTS), lambda i: (0, 0)),
            pl.BlockSpec((1, _N_SPECIES), lambda i: (0, 0)),
        ],
        out_specs=pl.BlockSpec((_NODE_BLOCK, 1), lambda i: (i, 0)),
        out_shape=jax.ShapeDtypeStruct((n, 1), jnp.float32),
    )(node_attrs, atomic_numbers.reshape(1, -1), covalent_radii.reshape(1, -1))


def _edge_body(rad_hbm, ei_hbm, x_hbm, out_hbm,
               rad_v, sidx_a, ridx_a, x_a, o_a, sidx_b, ridx_b, x_b, o_b,
               sem_in_a, sem_in_b, sem_out_a, sem_out_b):
    wid = lax.axis_index("s") * 2 + lax.axis_index("c")
    base_w = wid * _EDGE_PER_W

    def issue_in(g, sidx_v, ridx_v, x_v, sem):
        base = base_w + g * _CHUNK
        pltpu.async_copy(ei_hbm.at[pl.ds(base, _CHUNK)], sidx_v, sem)
        pltpu.async_copy(ei_hbm.at[pl.ds(_N_EDGES + base, _CHUNK)], ridx_v, sem)
        pltpu.async_copy(x_hbm.at[pl.ds(base, _CHUNK)], x_v, sem)

    def wait_in(sidx_v, ridx_v, x_v, sem):
        pltpu.make_async_copy(ei_hbm.at[pl.ds(base_w, _CHUNK)], sidx_v, sem).wait()
        pltpu.make_async_copy(ei_hbm.at[pl.ds(base_w, _CHUNK)], ridx_v, sem).wait()
        pltpu.make_async_copy(x_hbm.at[pl.ds(base_w, _CHUNK)], x_v, sem).wait()

    def wait_out(o_v, sem):
        pltpu.make_async_copy(o_v, out_hbm.at[pl.ds(base_w, _CHUNK)], sem).wait()

    def compute(sidx_v, ridx_v, x_v, o_v):
        @plsc.parallel_loop(0, _CHUNK, step=_L, unroll=_UNROLL)
        def step(i):
            si = sidx_v[pl.ds(i, _L)]
            ri = ridx_v[pl.ds(i, _L)]
            ra = plsc.load_gather(rad_v, [si])
            rb = plsc.load_gather(rad_v, [ri])
            r0 = ra + rb
            xv = x_v[pl.ds(i, _L)]
            t = (100.0 / 7.0) - (96.0 / 7.0) * (xv / r0)
            p0 = 0.75 * r0
            o_v[pl.ds(i, _L)] = p0 + (xv - p0) / (1.0 + jnp.exp(t))

    pltpu.sync_copy(rad_hbm, rad_v)
    issue_in(0, sidx_a, ridx_a, x_a, sem_in_a)
    issue_in(1, sidx_b, ridx_b, x_b, sem_in_b)

    def pair(k, carry):
        ga = 2 * k
        gb = 2 * k + 1
        wait_in(sidx_a, ridx_a, x_a, sem_in_a)

        @pl.when(k > 0)
        def _():
            wait_out(o_a, sem_out_a)

        compute(sidx_a, ridx_a, x_a, o_a)
        pltpu.async_copy(o_a, out_hbm.at[pl.ds(base_w + ga * _CHUNK, _CHUNK)],
                         sem_out_a)

        @pl.when(k < _NCHUNK // 2 - 1)
        def _():
            issue_in(ga + 2, sidx_a, ridx_a, x_a, sem_in_a)

        wait_in(sidx_b, ridx_b, x_b, sem_in_b)

        @pl.when(k > 0)
        def _():
            wait_out(o_b, sem_out_b)

        compute(sidx_b, ridx_b, x_b, o_b)
        pltpu.async_copy(o_b, out_hbm.at[pl.ds(base_w + gb * _CHUNK, _CHUNK)],
                         sem_out_b)

        @pl.when(k < _NCHUNK // 2 - 1)
        def _():
            issue_in(gb + 2, sidx_b, ridx_b, x_b, sem_in_b)

        return carry

    lax.fori_loop(0, _NCHUNK // 2, pair, 0)
    wait_out(o_a, sem_out_a)
    wait_out(o_b, sem_out_b)


def _edge_call(rad, edge_index, xf):
    mesh = plsc.VectorSubcoreMesh(core_axis_name="c", subcore_axis_name="s")
    return pl.kernel(
        _edge_body,
        out_type=jax.ShapeDtypeStruct((_N_EDGES,), jnp.float32),
        mesh=mesh,
        scratch_types=[
            pltpu.VMEM((_N_NODES,), jnp.float32),
            pltpu.VMEM((_CHUNK,), jnp.int32),
            pltpu.VMEM((_CHUNK,), jnp.int32),
            pltpu.VMEM((_CHUNK,), jnp.float32),
            pltpu.VMEM((_CHUNK,), jnp.float32),
            pltpu.VMEM((_CHUNK,), jnp.int32),
            pltpu.VMEM((_CHUNK,), jnp.int32),
            pltpu.VMEM((_CHUNK,), jnp.float32),
            pltpu.VMEM((_CHUNK,), jnp.float32),
            pltpu.SemaphoreType.DMA,
            pltpu.SemaphoreType.DMA,
            pltpu.SemaphoreType.DMA,
            pltpu.SemaphoreType.DMA,
        ],
        compiler_params=pltpu.CompilerParams(needs_layout_passes=False),
    )(rad, edge_index, xf)


def kernel(x, node_attrs, edge_index, atomic_numbers, covalent_radii):
    rad = _node_rad_call(node_attrs, atomic_numbers, covalent_radii)
    out = _edge_call(rad.reshape(-1), edge_index.reshape(-1), x.reshape(-1))
    return out.reshape(-1, 1)


# speedup vs baseline: 1094.5218x; 1.1346x over previous
import functools

import jax
import jax.numpy as jnp
from jax import lax
from jax.experimental import pallas as pl
from jax.experimental.pallas import tpu as pltpu
from jax.experimental.pallas import tpu_sc as plsc

_N_NODES = 100000
_N_EDGES = 6400000
_NUM_ELEMENTS = 10
_N_SPECIES = 119

_NODE_BLOCK = 10000

_NW = 32
_CHUNK = 2048
_NCG = _N_EDGES // _CHUNK
_SLOTS = (_NCG + _NW - 1) // _NW
_SLOTS += _SLOTS % 2
_PAIRS = _SLOTS // 2
_L = 16
_UNROLL = 8


def _node_rad_body(attrs_ref, an_ref, cr_ref, out_ref):
    attrs = attrs_ref[...]
    an = an_ref[...]
    cr = cr_ref[...]
    bn = attrs.shape[0]
    ei = lax.broadcasted_iota(jnp.int32, (bn, _NUM_ELEMENTS), 1)
    rowmax = jnp.max(attrs, axis=1, keepdims=True)
    first = jnp.min(
        jnp.where(attrs == rowmax, ei, _NUM_ELEMENTS), axis=1, keepdims=True
    )
    z = jnp.sum(jnp.where(ei == first, an, 0), axis=1, keepdims=True)
    zi = lax.broadcasted_iota(jnp.int32, (bn, _N_SPECIES), 1)
    rad = jnp.sum(jnp.where(zi == z, cr, 0.0), axis=1, keepdims=True)
    out_ref[...] = rad


def _node_rad_call(node_attrs, atomic_numbers, covalent_radii):
    n = node_attrs.shape[0]
    grid = n // _NODE_BLOCK
    return pl.pallas_call(
        _node_rad_body,
        grid=(grid,),
        in_specs=[
            pl.BlockSpec((_NODE_BLOCK, _NUM_ELEMENTS), lambda i: (i, 0)),
            pl.BlockSpec((1, _NUM_ELEMENTS), lambda i: (0, 0)),
            pl.BlockSpec((1, _N_SPECIES), lambda i: (0, 0)),
        ],
        out_specs=pl.BlockSpec((_NODE_BLOCK, 1), lambda i: (i, 0)),
        out_shape=jax.ShapeDtypeStruct((n, 1), jnp.float32),
    )(node_attrs, atomic_numbers.reshape(1, -1), covalent_radii.reshape(1, -1))


def _edge_body(rad_hbm, ei_hbm, x_hbm, out_hbm,
               rad_v, ei_a, x_a, o_a, ei_b, x_b, o_b,
               sem_in_a, sem_in_b, sem_out_a, sem_out_b):
    wid = lax.axis_index("s") * 2 + lax.axis_index("c")

    def valid(j):
        return j * _NW + wid < _NCG

    def cbase(j):
        return (j * _NW + wid) * _CHUNK

    def issue_in(j, ei_v, x_v, sem):
        @pl.when(valid(j))
        def _():
            base = cbase(j)
            pltpu.async_copy(ei_hbm.at[:, pl.ds(base, _CHUNK)], ei_v, sem)
            pltpu.async_copy(x_hbm.at[pl.ds(base, _CHUNK)], x_v, sem)

    def wait_in(j, ei_v, x_v, sem):
        @pl.when(valid(j))
        def _():
            pltpu.make_async_copy(ei_hbm.at[:, pl.ds(0, _CHUNK)], ei_v, sem).wait()
            pltpu.make_async_copy(x_hbm.at[pl.ds(0, _CHUNK)], x_v, sem).wait()

    def wait_out(cond, o_v, sem):
        @pl.when(cond)
        def _():
            pltpu.make_async_copy(o_v, out_hbm.at[pl.ds(0, _CHUNK)], sem).wait()

    def compute_and_out(j, ei_v, x_v, o_v, sem_out):
        @pl.when(valid(j))
        def _():
            @plsc.parallel_loop(0, _CHUNK, step=_L, unroll=_UNROLL)
            def step(i):
                si = ei_v[0, pl.ds(i, _L)]
                ri = ei_v[1, pl.ds(i, _L)]
                ra = plsc.load_gather(rad_v, [si])
                rb = plsc.load_gather(rad_v, [ri])
                r0 = ra + rb
                xv = x_v[pl.ds(i, _L)]
                t = (100.0 / 7.0) - (96.0 / 7.0) * (xv / r0)
                p0 = 0.75 * r0
                o_v[pl.ds(i, _L)] = p0 + (xv - p0) / (1.0 + jnp.exp(t))

            pltpu.async_copy(o_v, out_hbm.at[pl.ds(cbase(j), _CHUNK)], sem_out)

    pltpu.sync_copy(rad_hbm, rad_v)
    issue_in(0, ei_a, x_a, sem_in_a)
    issue_in(1, ei_b, x_b, sem_in_b)

    def pair(k, carry):
        ja = 2 * k
        jb = 2 * k + 1
        wait_in(ja, ei_a, x_a, sem_in_a)
        wait_out(jnp.logical_and(k > 0, valid(ja - 2)), o_a, sem_out_a)
        compute_and_out(ja, ei_a, x_a, o_a, sem_out_a)
        issue_in(ja + 2, ei_a, x_a, sem_in_a)
        wait_in(jb, ei_b, x_b, sem_in_b)
        wait_out(jnp.logical_and(k > 0, valid(jb - 2)), o_b, sem_out_b)
        compute_and_out(jb, ei_b, x_b, o_b, sem_out_b)
        issue_in(jb + 2, ei_b, x_b, sem_in_b)
        return carry

    lax.fori_loop(0, _PAIRS, pair, 0)
    wait_out(valid(_SLOTS - 2), o_a, sem_out_a)
    wait_out(valid(_SLOTS - 1), o_b, sem_out_b)


def _edge_call(rad, edge_index, xf):
    mesh = plsc.VectorSubcoreMesh(core_axis_name="c", subcore_axis_name="s")
    return pl.kernel(
        _edge_body,
        out_type=jax.ShapeDtypeStruct((_N_EDGES,), jnp.float32),
        mesh=mesh,
        scratch_types=[
            pltpu.VMEM((_N_NODES,), jnp.float32),
            pltpu.VMEM((2, _CHUNK), jnp.int32),
            pltpu.VMEM((_CHUNK,), jnp.float32),
            pltpu.VMEM((_CHUNK,), jnp.float32),
            pltpu.VMEM((2, _CHUNK), jnp.int32),
            pltpu.VMEM((_CHUNK,), jnp.float32),
            pltpu.VMEM((_CHUNK,), jnp.float32),
            pltpu.SemaphoreType.DMA,
            pltpu.SemaphoreType.DMA,
            pltpu.SemaphoreType.DMA,
            pltpu.SemaphoreType.DMA,
        ],
        compiler_params=pltpu.CompilerParams(needs_layout_passes=False),
    )(rad, edge_index, xf)


def kernel(x, node_attrs, edge_index, atomic_numbers, covalent_radii):
    rad = _node_rad_call(node_attrs, atomic_numbers, covalent_radii)
    out = _edge_call(rad.reshape(-1), edge_index, x.reshape(-1))
    return out.reshape(-1, 1)
